# baseline (speedup 1.0000x reference)
import jax
import jax.numpy as jnp
from jax.experimental import pallas as pl
from jax.experimental.pallas import tpu as pltpu

BLOCK_M = 512


def _router_block(h_top_ref, h_bot_ref, w_ref, out_ref):
    w = w_ref[...]

    def probs(h):
        logits = jax.lax.dot_general(
            h, w, (((1,), (1,)), ((), ())), preferred_element_type=jnp.float32
        )
        m = jnp.max(logits, axis=-1, keepdims=True)
        e = jnp.exp(logits - m)
        return e / jnp.sum(e, axis=-1, keepdims=True)

    out_ref[0] = probs(h_top_ref[...])
    out_ref[1] = probs(h_bot_ref[...])


def kernel(hidden_states, gate_weight):
    n_tokens, hidden = hidden_states.shape
    n_experts = gate_weight.shape[0]
    half_blocks = n_tokens // (2 * BLOCK_M)
    grid = (half_blocks,)
    out = pl.pallas_call(
        _router_block,
        grid=grid,
        in_specs=[
            pl.BlockSpec((BLOCK_M, hidden), lambda i: (i, 0)),
            pl.BlockSpec((BLOCK_M, hidden), lambda i, nb=half_blocks: (i + nb, 0)),
            pl.BlockSpec((n_experts, hidden), lambda i: (0, 0)),
        ],
        out_specs=pl.BlockSpec((2, BLOCK_M, n_experts), lambda i: (0, i, 0)),
        out_shape=jax.ShapeDtypeStruct((2, n_tokens // 2, n_experts), jnp.float32),
        compiler_params=pltpu.CompilerParams(
            dimension_semantics=("arbitrary",),
        ),
    )(hidden_states, hidden_states, gate_weight)
    return out.reshape(n_tokens, n_experts)
